# baseline (speedup 1.0000x reference)
import functools

import jax
import jax.numpy as jnp
from jax import lax
from jax.experimental import pallas as pl
from jax.experimental.pallas import tpu as pltpu
from jax.experimental.pallas import tpu_sc as plsc

_K = 1024
_D = 64
_B = 64
_P = 576
_N = _B * _P
_COST = 0.25

_NW = 32
_BPW = _N // _NW
_CH = 128
_NCH = _BPW // _CH


_P2 = 2 * _P


def _dist_body(zt_ref, wm2_ref, wsq_ref, codes_ref, loss_ref):
    ztc = jnp.concatenate([zt_ref[pl.ds(0, _D), :], zt_ref[pl.ds(_D, _D), :]],
                          axis=1)
    t2 = lax.dot_general(wm2_ref[...], ztc,
                         dimension_numbers=(((1,), (0,)), ((), ())),
                         preferred_element_type=jnp.float32)
    zsq = jnp.sum(ztc * ztc, axis=0, keepdims=True)
    dist = (zsq + wsq_ref[...]) + t2
    m = jnp.min(dist, axis=0, keepdims=True)
    ii = lax.broadcasted_iota(jnp.int32, dist.shape, 0).astype(jnp.float32)
    codes_f = jnp.min(jnp.where(dist == m, ii, jnp.float32(_K)), axis=0)
    codes_ref[0, 0, :] = codes_f.astype(jnp.int32)

    @pl.when(pl.program_id(0) == 0)
    def _init():
        loss_ref[...] = jnp.zeros_like(loss_ref)

    loss_ref[...] += jnp.sum(m).reshape(1, 1)


_GH = _B // 4
_dist_call = pl.pallas_call(
    _dist_body,
    grid=(_GH,),
    in_specs=[
        pl.BlockSpec((2 * _D, _P), lambda i: (i, 0)),
        pl.BlockSpec((_K, _D), lambda i: (0, 0)),
        pl.BlockSpec((_K, 1), lambda i: (0, 0)),
    ],
    out_specs=[
        pl.BlockSpec((1, 1, _P2), lambda i: (i, 0, 0)),
        pl.BlockSpec((1, 1), lambda i: (0, 0)),
    ],
    out_shape=[
        jax.ShapeDtypeStruct((_GH, 1, _P2), jnp.int32),
        jax.ShapeDtypeStruct((1, 1), jnp.float32),
    ],
)


_NH = _N // 2
_BPWH = _NH // _NW
_CHUNKS = [(0, _CH), (_CH, _CH), (2 * _CH, _CH), (3 * _CH, _CH),
           (4 * _CH, _BPWH - 4 * _CH)]


def _gather_body(w_hbm, codes_hbm, out_hbm, idx_v, rows_v, sem):
    wid = lax.axis_index("s") * 2 + lax.axis_index("c")
    pltpu.sync_copy(codes_hbm.at[pl.ds(wid * _BPWH, _BPWH)], idx_v)
    copies = [
        pltpu.async_copy(w_hbm.at[idx_v.at[pl.ds(off, n)]],
                         rows_v.at[pl.ds(off, n)], sem)
        for off, n in _CHUNKS
    ]
    for c in copies:
        c.wait()
    pltpu.sync_copy(rows_v, out_hbm.at[pl.ds(wid * _BPWH, _BPWH)])


def _gather_call(W, codes):
    run = functools.partial(
        pl.kernel,
        mesh=plsc.VectorSubcoreMesh(core_axis_name="c", subcore_axis_name="s"),
        out_type=jax.ShapeDtypeStruct((_NH, _D), jnp.float32),
        scratch_types=[
            pltpu.VMEM((_BPWH,), jnp.int32),
            pltpu.VMEM((_BPWH, _D), jnp.float32),
            pltpu.SemaphoreType.DMA,
        ],
        compiler_params=pltpu.CompilerParams(use_tc_tiling_on_sc=False),
    )(_gather_body)
    return run(W, codes)


def kernel(z, W):
    zt = jnp.swapaxes(z, 1, 2).reshape(_B * _D, _P)
    wsq = jnp.sum(W ** 2, axis=1, keepdims=True)
    wm2 = W * jnp.float32(-2.0)
    codes_a, loss_a = _dist_call(zt[: _NH // _P * _D, :], wm2, wsq)
    codes_b, loss_b = _dist_call(zt[_NH // _P * _D:, :], wm2, wsq)
    zq_a = _gather_call(W, codes_a.reshape(_NH))
    zq_b = _gather_call(W, codes_b.reshape(_NH))
    z_q = jnp.concatenate([zq_a, zq_b], axis=0)
    loss_acc = loss_a + loss_b
    vq_loss = (loss_acc[0, 0] / jnp.float32(_N * _D)) * jnp.float32(_COST)
    codes_out = jnp.concatenate(
        [codes_a.reshape(_B // 2, _P), codes_b.reshape(_B // 2, _P)], axis=0)
    return (vq_loss, z_q.reshape(z.shape), codes_out)


# speedup vs baseline: 1.1981x; 1.1981x over previous
import functools

import jax
import jax.numpy as jnp
from jax import lax
from jax.experimental import pallas as pl
from jax.experimental.pallas import tpu as pltpu
from jax.experimental.pallas import tpu_sc as plsc

_K = 1024
_D = 64
_B = 64
_P = 576
_N = _B * _P
_COST = 0.25

_NW = 32
_BPW = _N // _NW
_CH = 128
_NCH = _BPW // _CH


_P2 = 2 * _P


def _dist_body(zt_ref, wm2_ref, wsq_ref, codes_ref, loss_ref):
    ztc = jnp.concatenate([zt_ref[pl.ds(0, _D), :], zt_ref[pl.ds(_D, _D), :]],
                          axis=1)
    t2 = lax.dot_general(wm2_ref[...], ztc,
                         dimension_numbers=(((1,), (0,)), ((), ())),
                         preferred_element_type=jnp.float32)
    zsq = jnp.sum(ztc * ztc, axis=0, keepdims=True)
    dist = (zsq + wsq_ref[...]) + t2
    m = jnp.min(dist, axis=0, keepdims=True)
    ii = lax.broadcasted_iota(jnp.int32, dist.shape, 0).astype(jnp.float32)
    codes_f = jnp.min(jnp.where(dist == m, ii, jnp.float32(_K)), axis=0)
    codes_ref[0, 0, :] = codes_f.astype(jnp.int32)

    @pl.when(pl.program_id(0) == 0)
    def _init():
        loss_ref[...] = jnp.zeros_like(loss_ref)

    loss_ref[...] += jnp.sum(m).reshape(1, 1)


_dist_call = pl.pallas_call(
    _dist_body,
    grid=(_B // 2,),
    in_specs=[
        pl.BlockSpec((2 * _D, _P), lambda i: (i, 0)),
        pl.BlockSpec((_K, _D), lambda i: (0, 0)),
        pl.BlockSpec((_K, 1), lambda i: (0, 0)),
    ],
    out_specs=[
        pl.BlockSpec((1, 1, _P2), lambda i: (i, 0, 0)),
        pl.BlockSpec((1, 1), lambda i: (0, 0)),
    ],
    out_shape=[
        jax.ShapeDtypeStruct((_B // 2, 1, _P2), jnp.int32),
        jax.ShapeDtypeStruct((1, 1), jnp.float32),
    ],
)


def _gather_body(w_hbm, codes_hbm, out_hbm, idx_v, rows_v, sem):
    wid = lax.axis_index("s") * 2 + lax.axis_index("c")
    pltpu.sync_copy(codes_hbm.at[wid], idx_v)
    copies = [
        pltpu.async_copy(w_hbm.at[idx_v.at[j]],
                         rows_v.at[pl.ds(j * _CH, _CH)], sem)
        for j in range(_NCH)
    ]
    for c in copies:
        c.wait()
    pltpu.sync_copy(rows_v, out_hbm.at[pl.ds(wid * _BPW, _BPW)])


def _gather_call(W, codes2d):
    run = functools.partial(
        pl.kernel,
        mesh=plsc.VectorSubcoreMesh(core_axis_name="c", subcore_axis_name="s"),
        out_type=jax.ShapeDtypeStruct((_N, _D), jnp.float32),
        scratch_types=[
            pltpu.VMEM((_NCH, _CH), jnp.int32),
            pltpu.VMEM((_BPW, _D), jnp.float32),
            pltpu.SemaphoreType.DMA,
        ],
        compiler_params=pltpu.CompilerParams(use_tc_tiling_on_sc=False),
    )(_gather_body)
    return run(W, codes2d)


def kernel(z, W):
    zt = jnp.swapaxes(z, 1, 2).reshape(_B * _D, _P)
    wsq = jnp.sum(W ** 2, axis=1, keepdims=True)
    codes3d, loss_acc = _dist_call(zt, W * jnp.float32(-2.0), wsq)
    codes = codes3d.reshape(_N)
    z_q = _gather_call(W, codes.reshape(_NW, _NCH, _CH))
    vq_loss = (loss_acc[0, 0] / jnp.float32(_N * _D)) * jnp.float32(_COST)
    return (vq_loss, z_q.reshape(z.shape), codes3d.reshape(_B, _P))
